# baseline (speedup 1.0000x reference)
import functools

import jax
import jax.numpy as jnp
from jax import lax
from jax.experimental import pallas as pl
from jax.experimental.pallas import tpu as pltpu
from jax.experimental.pallas import tpu_sc as plsc

VOCAB = 1_000_000
DIM = 64
BATCH = 4096
SEQ = 200

_INFO = plsc.get_sparse_core_info()
_NC = _INFO.num_cores
_NS = _INFO.num_subcores
_NW = _NC * _NS

_B = BATCH * SEQ
_IW = 128
_ROWS_PER_W = _B // _NW
_IDX_ROWS_PER_W = _ROWS_PER_W // _IW

_CHUNK_STREAMS = 4
_CHUNK = _IW * _CHUNK_STREAMS
_NCHUNKS = _ROWS_PER_W // _CHUNK


def _gather_kernel(table_hbm, idx_hbm, out_hbm, idx_v, rows_v, gsem, osem):
    wid = lax.axis_index("s") * _NC + lax.axis_index("c")
    idx_base = wid * _IDX_ROWS_PER_W
    row_base = wid * _ROWS_PER_W

    pltpu.sync_copy(idx_hbm.at[pl.ds(idx_base, _IDX_ROWS_PER_W)], idx_v)

    def chunk_body(c, carry):
        buf = lax.rem(c, 2)
        handles = []
        for i in range(_CHUNK_STREAMS):
            h = pltpu.async_copy(
                table_hbm.at[idx_v.at[c * _CHUNK_STREAMS + i]],
                rows_v.at[buf, pl.ds(i * _IW, _IW)],
                gsem,
            )
            handles.append(h)
        for h in handles:
            h.wait()
        pltpu.async_copy(
            rows_v.at[buf],
            out_hbm.at[pl.ds(row_base + c * _CHUNK, _CHUNK)],
            osem,
        ).wait()
        return carry

    lax.fori_loop(0, _NCHUNKS, chunk_body, 0)


@functools.partial(jax.jit, donate_argnums=())
def kernel(text, img, table):
    del img
    idx2d = text.reshape(_B // _IW, _IW)

    mesh = plsc.VectorSubcoreMesh(core_axis_name="c", subcore_axis_name="s")
    out_flat = pl.kernel(
        _gather_kernel,
        out_type=jax.ShapeDtypeStruct((_B, DIM), jnp.float32),
        mesh=mesh,
        scratch_types=[
            pltpu.VMEM((_IDX_ROWS_PER_W, _IW), jnp.int32),
            pltpu.VMEM((2, _CHUNK, DIM), jnp.float32),
            pltpu.SemaphoreType.DMA,
            pltpu.SemaphoreType.DMA,
        ],
        compiler_params=pltpu.CompilerParams(use_tc_tiling_on_sc=False),
    )(table, idx2d)
    return out_flat.reshape(BATCH, SEQ, DIM)


# speedup vs baseline: 1.0236x; 1.0236x over previous
import functools

import jax
import jax.numpy as jnp
from jax import lax
from jax.experimental import pallas as pl
from jax.experimental.pallas import tpu as pltpu
from jax.experimental.pallas import tpu_sc as plsc

VOCAB = 1_000_000
DIM = 64
BATCH = 4096
SEQ = 200

_INFO = plsc.get_sparse_core_info()
_NC = _INFO.num_cores
_NS = _INFO.num_subcores
_NW = _NC * _NS

_B = BATCH * SEQ
_IW = 128
_ROWS_PER_W = _B // _NW
_IDX_ROWS_PER_W = _ROWS_PER_W // _IW

_CHUNK_STREAMS = 2
_CHUNK = _IW * _CHUNK_STREAMS
_NBUF = 5
_SUPER = _CHUNK * _NBUF
_NSUPER = _ROWS_PER_W // _SUPER


def _gather_kernel(table_hbm, idx_hbm, out_hbm, idx_v, rows_v, gsem, osem):
    wid = lax.axis_index("s") * _NC + lax.axis_index("c")
    idx_base = wid * _IDX_ROWS_PER_W
    row_base = wid * _ROWS_PER_W

    pltpu.sync_copy(idx_hbm.at[pl.ds(idx_base, _IDX_ROWS_PER_W)], idx_v)

    irows_per_super = _SUPER // _IW

    def fire_gathers(s, b):
        for i in range(_CHUNK_STREAMS):
            pltpu.async_copy(
                table_hbm.at[idx_v.at[s * irows_per_super
                                      + b * _CHUNK_STREAMS + i]],
                rows_v.at[b, pl.ds(i * _IW, _IW)],
                gsem.at[b],
            )

    def wait_gathers(b):
        pltpu.make_async_copy(
            table_hbm.at[pl.ds(0, _CHUNK)], rows_v.at[b], gsem.at[b],
        ).wait()

    def fire_out(s, b):
        pltpu.async_copy(
            rows_v.at[b],
            out_hbm.at[pl.ds(row_base + s * _SUPER + b * _CHUNK, _CHUNK)],
            osem.at[b],
        )

    def wait_out(b):
        pltpu.make_async_copy(
            table_hbm.at[pl.ds(0, _CHUNK)], rows_v.at[b], osem.at[b],
        ).wait()

    for b in range(_NBUF):
        fire_gathers(0, b)

    def super_body(s, carry):
        for b in range(_NBUF):
            wait_gathers(b)
            fire_out(s, b)
        for b in range(_NBUF):
            wait_out(b)
            fire_gathers(s + 1, b)
        return carry

    lax.fori_loop(0, _NSUPER - 1, super_body, 0)

    for b in range(_NBUF):
        wait_gathers(b)
        fire_out(_NSUPER - 1, b)
    for b in range(_NBUF):
        wait_out(b)


@functools.partial(jax.jit, donate_argnums=())
def kernel(text, img, table):
    del img
    idx2d = text.reshape(_B // _IW, _IW)

    mesh = plsc.VectorSubcoreMesh(core_axis_name="c", subcore_axis_name="s")
    out_flat = pl.kernel(
        _gather_kernel,
        out_type=jax.ShapeDtypeStruct((_B, DIM), jnp.float32),
        mesh=mesh,
        scratch_types=[
            pltpu.VMEM((_IDX_ROWS_PER_W, _IW), jnp.int32),
            pltpu.VMEM((_NBUF, _CHUNK, DIM), jnp.float32),
            pltpu.SemaphoreType.DMA((_NBUF,)),
            pltpu.SemaphoreType.DMA((_NBUF,)),
        ],
        compiler_params=pltpu.CompilerParams(use_tc_tiling_on_sc=False),
    )(table, idx2d)
    return out_flat.reshape(BATCH, SEQ, DIM)
